# Initial kernel scaffold; baseline (speedup 1.0000x reference)
import jax
import jax.numpy as jnp
from jax.experimental import pallas as pl


def kernel(x, edge_index, edge_attr, W_l, W_r, W_e, att, bias, ln_gamma, ln_beta, mlp_W, mlp_b):
    raise NotImplementedError("write your pallas kernel here")


# speedup vs baseline: 7.7340x; 7.7340x over previous
import functools

import jax
import jax.numpy as jnp
from jax import lax
from jax.experimental import pallas as pl
from jax.experimental.pallas import tpu as pltpu
from jax.experimental.pallas import tpu_sc as plsc

F32 = jnp.float32

N = 10000
E = 320000
HEADS = 4
C = 32
D = 128
NEG_SLOPE = 0.2

BN = 1000
BE = 2000


def _k1_body(x_ref, wl_ref, wr_ref, xl_ref, xr_ref):
    xv = x_ref[...]
    xl_ref[...] = jnp.dot(xv, wl_ref[...], preferred_element_type=F32)
    xr_ref[...] = jnp.dot(xv, wr_ref[...], preferred_element_type=F32)


def _project_nodes(x, W_l, W_r):
    return pl.pallas_call(
        _k1_body,
        grid=(N // BN,),
        in_specs=[
            pl.BlockSpec((BN, D), lambda i: (i, 0)),
            pl.BlockSpec((D, D), lambda i: (0, 0)),
            pl.BlockSpec((D, D), lambda i: (0, 0)),
        ],
        out_specs=[pl.BlockSpec((BN, D), lambda i: (i, 0))] * 2,
        out_shape=[jax.ShapeDtypeStruct((N, D), F32)] * 2,
    )(x, W_l, W_r)


def _k3_body(ea_ref, gxl_ref, gxr_ref, we_ref, attf_ref, sh_ref, mask_ref,
             r16_ref, m_ref, ex_ref):
    e = jnp.dot(ea_ref[...], we_ref[...], preferred_element_type=F32)
    s = gxl_ref[...] + gxr_ref[...] + e
    h = jnp.where(s >= 0, s, NEG_SLOPE * s)
    hm = h * attf_ref[...]
    lp = jnp.dot(hm, sh_ref[...], preferred_element_type=F32)
    ex = jnp.exp(lp) * mask_ref[...]
    ex_ref[...] = ex
    exr = jnp.dot(ex, r16_ref[...], preferred_element_type=F32)
    m_ref[...] = exr * gxl_ref[...]


def _edge_messages(edge_attr, gxl, gxr, W_e, att_flat, Sh, mask16, R16):
    return pl.pallas_call(
        _k3_body,
        grid=(E // BE,),
        in_specs=[
            pl.BlockSpec((BE, D), lambda i: (i, 0)),
            pl.BlockSpec((BE, D), lambda i: (i, 0)),
            pl.BlockSpec((BE, D), lambda i: (i, 0)),
            pl.BlockSpec((D, D), lambda i: (0, 0)),
            pl.BlockSpec((1, D), lambda i: (0, 0)),
            pl.BlockSpec((D, 16), lambda i: (0, 0)),
            pl.BlockSpec((1, 16), lambda i: (0, 0)),
            pl.BlockSpec((16, D), lambda i: (0, 0)),
        ],
        out_specs=[
            pl.BlockSpec((BE, D), lambda i: (i, 0)),
            pl.BlockSpec((BE, 16), lambda i: (i, 0)),
        ],
        out_shape=[
            jax.ShapeDtypeStruct((E, D), F32),
            jax.ShapeDtypeStruct((E, 16), F32),
        ],
    )(edge_attr, gxl, gxr, W_e, att_flat, Sh, mask16, R16)


def _k6_body(mp_ref, dp_ref, r16_ref, bias_ref, out_ref):
    m = mp_ref[0] + mp_ref[1]
    d = dp_ref[0] + dp_ref[1]
    drep = jnp.dot(d, r16_ref[...], preferred_element_type=F32)
    out_ref[...] = m / (drep + 1e-16) + bias_ref[...]


def _normalize_nodes(mP, dP, R16, bias_row):
    return pl.pallas_call(
        _k6_body,
        grid=(N // BN,),
        in_specs=[
            pl.BlockSpec((2, BN, D), lambda i: (0, i, 0)),
            pl.BlockSpec((2, BN, 16), lambda i: (0, i, 0)),
            pl.BlockSpec((16, D), lambda i: (0, 0)),
            pl.BlockSpec((1, D), lambda i: (0, 0)),
        ],
        out_specs=pl.BlockSpec((BN, D), lambda i: (i, 0)),
        out_shape=jax.ShapeDtypeStruct((N, D), F32),
    )(mP, dP, R16, bias_row)


def _k8_body(gos_ref, god_ref, ea_ref, wt_ref, wb_ref, gs_ref, gd_ref,
             bs_ref, bd_ref, mb_ref, out_ref):
    gos = gos_ref[...]
    god = god_ref[...]
    mu = (jnp.sum(gos, axis=1, keepdims=True)
          + jnp.sum(god, axis=1, keepdims=True)) / (2 * D)
    cs = gos - mu
    cd = god - mu
    var = (jnp.sum(cs * cs, axis=1, keepdims=True)
           + jnp.sum(cd * cd, axis=1, keepdims=True)) / (2 * D)
    rstd = lax.rsqrt(var + 1e-5)
    ns = cs * rstd * gs_ref[...] + bs_ref[...]
    nd = cd * rstd * gd_ref[...] + bd_ref[...]
    ns = jnp.maximum(ns, 0.0)
    nd = jnp.maximum(nd, 0.0)
    delta = (jnp.dot(ns, wt_ref[...], preferred_element_type=F32)
             + jnp.dot(nd, wb_ref[...], preferred_element_type=F32)
             + mb_ref[...])
    out_ref[...] = ea_ref[...] + delta


def _edge_mlp(gos, god, edge_attr, Wt, Wb, gs, gd, bs, bd, mb):
    return pl.pallas_call(
        _k8_body,
        grid=(E // BE,),
        in_specs=[
            pl.BlockSpec((BE, D), lambda i: (i, 0)),
            pl.BlockSpec((BE, D), lambda i: (i, 0)),
            pl.BlockSpec((BE, D), lambda i: (i, 0)),
            pl.BlockSpec((D, D), lambda i: (0, 0)),
            pl.BlockSpec((D, D), lambda i: (0, 0)),
            pl.BlockSpec((1, D), lambda i: (0, 0)),
            pl.BlockSpec((1, D), lambda i: (0, 0)),
            pl.BlockSpec((1, D), lambda i: (0, 0)),
            pl.BlockSpec((1, D), lambda i: (0, 0)),
            pl.BlockSpec((1, D), lambda i: (0, 0)),
        ],
        out_specs=pl.BlockSpec((BE, D), lambda i: (i, 0)),
        out_shape=jax.ShapeDtypeStruct((E, D), F32),
    )(gos, god, edge_attr, Wt, Wb, gs, gd, bs, bd, mb)


def _gather2(tableA, idxA, tableB, idxB):
    return jnp.take(tableA, idxA, axis=0), jnp.take(tableB, idxB, axis=0)


def _scatter_accumulate(m, ex, dst):
    msum = jax.ops.segment_sum(m, dst, num_segments=N)
    dsum = jax.ops.segment_sum(ex, dst, num_segments=N)
    mP = jnp.stack([msum, jnp.zeros_like(msum)])
    dP = jnp.stack([dsum, jnp.zeros_like(dsum)])
    return mP, dP


def kernel(x, edge_index, edge_attr, W_l, W_r, W_e, att, bias,
           ln_gamma, ln_beta, mlp_W, mlp_b):
    src = edge_index[0].astype(jnp.int32)
    dst = edge_index[1].astype(jnp.int32)

    att_flat = att.reshape(1, D).astype(F32)
    cc = jnp.arange(D, dtype=jnp.int32)
    hh = jnp.arange(16, dtype=jnp.int32)
    Sh = (cc[:, None] // C == hh[None, :]).astype(F32)
    mask16 = (hh < HEADS).astype(F32).reshape(1, 16)
    R16 = (hh[:, None] == cc[None, :] // C).astype(F32)
    bias_row = bias.reshape(1, D).astype(F32)
    gs = ln_gamma[:D].reshape(1, D).astype(F32)
    gd = ln_gamma[D:].reshape(1, D).astype(F32)
    bs = ln_beta[:D].reshape(1, D).astype(F32)
    bd = ln_beta[D:].reshape(1, D).astype(F32)
    Wt = mlp_W[:D].astype(F32)
    Wb = mlp_W[D:].astype(F32)
    mb = mlp_b.reshape(1, D).astype(F32)

    xl, xr = _project_nodes(x, W_l, W_r)
    gxl, gxr = _gather2(xl, src, xr, dst)
    m, ex = _edge_messages(edge_attr, gxl, gxr, W_e, att_flat, Sh, mask16, R16)
    mP, dP = _scatter_accumulate(m, ex, dst)
    out = _normalize_nodes(mP, dP, R16, bias_row)
    gos, god = _gather2(out, src, out, dst)
    new_edge_attr = _edge_mlp(gos, god, edge_attr, Wt, Wb, gs, gd, bs, bd, mb)
    return out, new_edge_attr


# speedup vs baseline: 15.5736x; 2.0136x over previous
import functools

import jax
import jax.numpy as jnp
from jax import lax
from jax.experimental import pallas as pl
from jax.experimental.pallas import tpu as pltpu
from jax.experimental.pallas import tpu_sc as plsc

F32 = jnp.float32

N = 10000
E = 320000
HEADS = 4
C = 32
D = 128
NEG_SLOPE = 0.2

BN = 1000
BE = 2000


def _k1_body(x_ref, wl_ref, wr_ref, xl_ref, xr_ref):
    xv = x_ref[...]
    xl_ref[...] = jnp.dot(xv, wl_ref[...], preferred_element_type=F32)
    xr_ref[...] = jnp.dot(xv, wr_ref[...], preferred_element_type=F32)


def _project_nodes(x, W_l, W_r):
    return pl.pallas_call(
        _k1_body,
        grid=(N // BN,),
        in_specs=[
            pl.BlockSpec((BN, D), lambda i: (i, 0)),
            pl.BlockSpec((D, D), lambda i: (0, 0)),
            pl.BlockSpec((D, D), lambda i: (0, 0)),
        ],
        out_specs=[pl.BlockSpec((BN, D), lambda i: (i, 0))] * 2,
        out_shape=[jax.ShapeDtypeStruct((N, D), F32)] * 2,
    )(x, W_l, W_r)


def _k3_body(ea_ref, gxl_ref, gxr_ref, we_ref, attf_ref, sh_ref, mask_ref,
             r16_ref, m_ref, ex_ref):
    e = jnp.dot(ea_ref[...], we_ref[...], preferred_element_type=F32)
    s = gxl_ref[...] + gxr_ref[...] + e
    h = jnp.where(s >= 0, s, NEG_SLOPE * s)
    hm = h * attf_ref[...]
    lp = jnp.dot(hm, sh_ref[...], preferred_element_type=F32)
    ex = jnp.exp(lp) * mask_ref[...]
    ex_ref[...] = ex
    exr = jnp.dot(ex, r16_ref[...], preferred_element_type=F32)
    m_ref[...] = exr * gxl_ref[...]


def _edge_messages(edge_attr, gxl, gxr, W_e, att_flat, Sh, mask16, R16):
    return pl.pallas_call(
        _k3_body,
        grid=(E // BE,),
        in_specs=[
            pl.BlockSpec((BE, D), lambda i: (i, 0)),
            pl.BlockSpec((BE, D), lambda i: (i, 0)),
            pl.BlockSpec((BE, D), lambda i: (i, 0)),
            pl.BlockSpec((D, D), lambda i: (0, 0)),
            pl.BlockSpec((1, D), lambda i: (0, 0)),
            pl.BlockSpec((D, 16), lambda i: (0, 0)),
            pl.BlockSpec((1, 16), lambda i: (0, 0)),
            pl.BlockSpec((16, D), lambda i: (0, 0)),
        ],
        out_specs=[
            pl.BlockSpec((BE, D), lambda i: (i, 0)),
            pl.BlockSpec((BE, 16), lambda i: (i, 0)),
        ],
        out_shape=[
            jax.ShapeDtypeStruct((E, D), F32),
            jax.ShapeDtypeStruct((E, 16), F32),
        ],
    )(edge_attr, gxl, gxr, W_e, att_flat, Sh, mask16, R16)


def _k6_body(mp_ref, dp_ref, r16_ref, bias_ref, out_ref):
    m = mp_ref[0] + mp_ref[1]
    d = dp_ref[0] + dp_ref[1]
    drep = jnp.dot(d, r16_ref[...], preferred_element_type=F32)
    out_ref[...] = m / (drep + 1e-16) + bias_ref[...]


def _normalize_nodes(mP, dP, R16, bias_row):
    return pl.pallas_call(
        _k6_body,
        grid=(N // BN,),
        in_specs=[
            pl.BlockSpec((2, BN, D), lambda i: (0, i, 0)),
            pl.BlockSpec((2, BN, 16), lambda i: (0, i, 0)),
            pl.BlockSpec((16, D), lambda i: (0, 0)),
            pl.BlockSpec((1, D), lambda i: (0, 0)),
        ],
        out_specs=pl.BlockSpec((BN, D), lambda i: (i, 0)),
        out_shape=jax.ShapeDtypeStruct((N, D), F32),
    )(mP, dP, R16, bias_row)


def _k8_body(gos_ref, god_ref, ea_ref, wt_ref, wb_ref, gs_ref, gd_ref,
             bs_ref, bd_ref, mb_ref, out_ref):
    gos = gos_ref[...]
    god = god_ref[...]
    mu = (jnp.sum(gos, axis=1, keepdims=True)
          + jnp.sum(god, axis=1, keepdims=True)) / (2 * D)
    cs = gos - mu
    cd = god - mu
    var = (jnp.sum(cs * cs, axis=1, keepdims=True)
           + jnp.sum(cd * cd, axis=1, keepdims=True)) / (2 * D)
    rstd = lax.rsqrt(var + 1e-5)
    ns = cs * rstd * gs_ref[...] + bs_ref[...]
    nd = cd * rstd * gd_ref[...] + bd_ref[...]
    ns = jnp.maximum(ns, 0.0)
    nd = jnp.maximum(nd, 0.0)
    delta = (jnp.dot(ns, wt_ref[...], preferred_element_type=F32)
             + jnp.dot(nd, wb_ref[...], preferred_element_type=F32)
             + mb_ref[...])
    out_ref[...] = ea_ref[...] + delta


def _edge_mlp(gos, god, edge_attr, Wt, Wb, gs, gd, bs, bd, mb):
    return pl.pallas_call(
        _k8_body,
        grid=(E // BE,),
        in_specs=[
            pl.BlockSpec((BE, D), lambda i: (i, 0)),
            pl.BlockSpec((BE, D), lambda i: (i, 0)),
            pl.BlockSpec((BE, D), lambda i: (i, 0)),
            pl.BlockSpec((D, D), lambda i: (0, 0)),
            pl.BlockSpec((D, D), lambda i: (0, 0)),
            pl.BlockSpec((1, D), lambda i: (0, 0)),
            pl.BlockSpec((1, D), lambda i: (0, 0)),
            pl.BlockSpec((1, D), lambda i: (0, 0)),
            pl.BlockSpec((1, D), lambda i: (0, 0)),
            pl.BlockSpec((1, D), lambda i: (0, 0)),
        ],
        out_specs=pl.BlockSpec((BE, D), lambda i: (i, 0)),
        out_shape=jax.ShapeDtypeStruct((E, D), F32),
    )(gos, god, edge_attr, Wt, Wb, gs, gd, bs, bd, mb)


NC = 2
NS = 16
NW = NC * NS
EPW = E // NW
GCH = 80
NGC = EPW // GCH

_sc_mesh = plsc.VectorSubcoreMesh(core_axis_name="c", subcore_axis_name="s")


def _gather2_body(tA, iA, tB, iB, oA, oB, liA, liB, bA, bB, gsem, wsem):
    wid = lax.axis_index("s") * NC + lax.axis_index("c")
    base = wid * EPW
    pltpu.sync_copy(iA.at[pl.ds(base, EPW)], liA)
    pltpu.sync_copy(iB.at[pl.ds(base, EPW)], liB)

    def body(ci, carry):
        off = ci * GCH
        cpA = pltpu.make_async_copy(tA.at[liA.at[pl.ds(off, GCH)]], bA, gsem)
        cpB = pltpu.make_async_copy(tB.at[liB.at[pl.ds(off, GCH)]], bB, gsem)
        cpA.start()
        cpB.start()
        cpA.wait()
        cpB.wait()
        wA = pltpu.make_async_copy(bA, oA.at[pl.ds(base + off, GCH)], wsem)
        wB = pltpu.make_async_copy(bB, oB.at[pl.ds(base + off, GCH)], wsem)
        wA.start()
        wB.start()
        wA.wait()
        wB.wait()
        return carry

    lax.fori_loop(0, NGC, body, 0)


def _gather2(tableA, idxA, tableB, idxB):
    f = functools.partial(
        pl.kernel,
        out_type=[jax.ShapeDtypeStruct((E, D), F32)] * 2,
        mesh=_sc_mesh,
        scratch_types=[
            pltpu.VMEM((EPW,), jnp.int32),
            pltpu.VMEM((EPW,), jnp.int32),
            pltpu.VMEM((GCH, D), F32),
            pltpu.VMEM((GCH, D), F32),
            pltpu.SemaphoreType.DMA,
            pltpu.SemaphoreType.DMA,
        ],
    )(_gather2_body)
    return f(tableA, idxA, tableB, idxB)


def _scatter_accumulate(m, ex, dst):
    msum = jax.ops.segment_sum(m, dst, num_segments=N)
    dsum = jax.ops.segment_sum(ex, dst, num_segments=N)
    mP = jnp.stack([msum, jnp.zeros_like(msum)])
    dP = jnp.stack([dsum, jnp.zeros_like(dsum)])
    return mP, dP


def kernel(x, edge_index, edge_attr, W_l, W_r, W_e, att, bias,
           ln_gamma, ln_beta, mlp_W, mlp_b):
    src = edge_index[0].astype(jnp.int32)
    dst = edge_index[1].astype(jnp.int32)

    att_flat = att.reshape(1, D).astype(F32)
    cc = jnp.arange(D, dtype=jnp.int32)
    hh = jnp.arange(16, dtype=jnp.int32)
    Sh = (cc[:, None] // C == hh[None, :]).astype(F32)
    mask16 = (hh < HEADS).astype(F32).reshape(1, 16)
    R16 = (hh[:, None] == cc[None, :] // C).astype(F32)
    bias_row = bias.reshape(1, D).astype(F32)
    gs = ln_gamma[:D].reshape(1, D).astype(F32)
    gd = ln_gamma[D:].reshape(1, D).astype(F32)
    bs = ln_beta[:D].reshape(1, D).astype(F32)
    bd = ln_beta[D:].reshape(1, D).astype(F32)
    Wt = mlp_W[:D].astype(F32)
    Wb = mlp_W[D:].astype(F32)
    mb = mlp_b.reshape(1, D).astype(F32)

    xl, xr = _project_nodes(x, W_l, W_r)
    gxl, gxr = _gather2(xl, src, xr, dst)
    m, ex = _edge_messages(edge_attr, gxl, gxr, W_e, att_flat, Sh, mask16, R16)
    mP, dP = _scatter_accumulate(m, ex, dst)
    out = _normalize_nodes(mP, dP, R16, bias_row)
    gos, god = _gather2(out, src, out, dst)
    new_edge_attr = _edge_mlp(gos, god, edge_attr, Wt, Wb, gs, gd, bs, bd, mb)
    return out, new_edge_attr


# speedup vs baseline: 18.3669x; 1.1794x over previous
import functools

import jax
import jax.numpy as jnp
from jax import lax
from jax.experimental import pallas as pl
from jax.experimental.pallas import tpu as pltpu
from jax.experimental.pallas import tpu_sc as plsc

F32 = jnp.float32

N = 10000
E = 320000
HEADS = 4
C = 32
D = 128
NEG_SLOPE = 0.2

BN = 1000
BE = 2000


def _k1_body(x_ref, wl_ref, wr_ref, xl_ref, xr_ref):
    xv = x_ref[...]
    xl_ref[...] = jnp.dot(xv, wl_ref[...], preferred_element_type=F32)
    xr_ref[...] = jnp.dot(xv, wr_ref[...], preferred_element_type=F32)


def _project_nodes(x, W_l, W_r):
    return pl.pallas_call(
        _k1_body,
        grid=(N // BN,),
        in_specs=[
            pl.BlockSpec((BN, D), lambda i: (i, 0)),
            pl.BlockSpec((D, D), lambda i: (0, 0)),
            pl.BlockSpec((D, D), lambda i: (0, 0)),
        ],
        out_specs=[pl.BlockSpec((BN, D), lambda i: (i, 0))] * 2,
        out_shape=[jax.ShapeDtypeStruct((N, D), F32)] * 2,
    )(x, W_l, W_r)


def _k3_body(ea_ref, gxl_ref, gxr_ref, we_ref, attf_ref, sh_ref, mask_ref,
             r16_ref, m_ref, exr_ref):
    e = jnp.dot(ea_ref[...], we_ref[...], preferred_element_type=F32)
    s = gxl_ref[...] + gxr_ref[...] + e
    h = jnp.where(s >= 0, s, NEG_SLOPE * s)
    hm = h * attf_ref[...]
    lp = jnp.dot(hm, sh_ref[...], preferred_element_type=F32)
    ex = jnp.exp(lp) * mask_ref[...]
    exr = jnp.dot(ex, r16_ref[...], preferred_element_type=F32)
    exr_ref[...] = exr
    m_ref[...] = exr * gxl_ref[...]


def _edge_messages(edge_attr, gxl, gxr, W_e, att_flat, Sh, mask16, R16):
    return pl.pallas_call(
        _k3_body,
        grid=(E // BE,),
        in_specs=[
            pl.BlockSpec((BE, D), lambda i: (i, 0)),
            pl.BlockSpec((BE, D), lambda i: (i, 0)),
            pl.BlockSpec((BE, D), lambda i: (i, 0)),
            pl.BlockSpec((D, D), lambda i: (0, 0)),
            pl.BlockSpec((1, D), lambda i: (0, 0)),
            pl.BlockSpec((D, 16), lambda i: (0, 0)),
            pl.BlockSpec((1, 16), lambda i: (0, 0)),
            pl.BlockSpec((16, D), lambda i: (0, 0)),
        ],
        out_specs=[
            pl.BlockSpec((BE, D), lambda i: (i, 0)),
            pl.BlockSpec((BE, D), lambda i: (i, 0)),
        ],
        out_shape=[
            jax.ShapeDtypeStruct((E, D), F32),
            jax.ShapeDtypeStruct((E, D), F32),
        ],
    )(edge_attr, gxl, gxr, W_e, att_flat, Sh, mask16, R16)


def _k6_body(mp_ref, ep_ref, bias_ref, out_ref):
    out_ref[...] = mp_ref[0] / (ep_ref[0] + 1e-16) + bias_ref[...]


def _normalize_nodes(mP, eP, bias_row):
    return pl.pallas_call(
        _k6_body,
        grid=(16,),
        in_specs=[
            pl.BlockSpec((1, 640, D), lambda i: (i // 8, i % 8, 0)),
            pl.BlockSpec((1, 640, D), lambda i: (i // 8, i % 8, 0)),
            pl.BlockSpec((1, D), lambda i: (0, 0)),
        ],
        out_specs=pl.BlockSpec((640, D), lambda i: (i, 0)),
        out_shape=jax.ShapeDtypeStruct((10240, D), F32),
    )(mP, eP, bias_row)


def _k8_body(gos_ref, god_ref, ea_ref, wt_ref, wb_ref, gs_ref, gd_ref,
             bs_ref, bd_ref, mb_ref, out_ref):
    gos = gos_ref[...]
    god = god_ref[...]
    mu = (jnp.sum(gos, axis=1, keepdims=True)
          + jnp.sum(god, axis=1, keepdims=True)) / (2 * D)
    cs = gos - mu
    cd = god - mu
    var = (jnp.sum(cs * cs, axis=1, keepdims=True)
           + jnp.sum(cd * cd, axis=1, keepdims=True)) / (2 * D)
    rstd = lax.rsqrt(var + 1e-5)
    ns = cs * rstd * gs_ref[...] + bs_ref[...]
    nd = cd * rstd * gd_ref[...] + bd_ref[...]
    ns = jnp.maximum(ns, 0.0)
    nd = jnp.maximum(nd, 0.0)
    delta = (jnp.dot(ns, wt_ref[...], preferred_element_type=F32)
             + jnp.dot(nd, wb_ref[...], preferred_element_type=F32)
             + mb_ref[...])
    out_ref[...] = ea_ref[...] + delta


def _edge_mlp(gos, god, edge_attr, Wt, Wb, gs, gd, bs, bd, mb):
    return pl.pallas_call(
        _k8_body,
        grid=(E // BE,),
        in_specs=[
            pl.BlockSpec((BE, D), lambda i: (i, 0)),
            pl.BlockSpec((BE, D), lambda i: (i, 0)),
            pl.BlockSpec((BE, D), lambda i: (i, 0)),
            pl.BlockSpec((D, D), lambda i: (0, 0)),
            pl.BlockSpec((D, D), lambda i: (0, 0)),
            pl.BlockSpec((1, D), lambda i: (0, 0)),
            pl.BlockSpec((1, D), lambda i: (0, 0)),
            pl.BlockSpec((1, D), lambda i: (0, 0)),
            pl.BlockSpec((1, D), lambda i: (0, 0)),
            pl.BlockSpec((1, D), lambda i: (0, 0)),
        ],
        out_specs=pl.BlockSpec((BE, D), lambda i: (i, 0)),
        out_shape=jax.ShapeDtypeStruct((E, D), F32),
    )(gos, god, edge_attr, Wt, Wb, gs, gd, bs, bd, mb)


NC = 2
NS = 16
NW = NC * NS
EPW = E // NW
GCH = 80
NGC = EPW // GCH

_sc_mesh = plsc.VectorSubcoreMesh(core_axis_name="c", subcore_axis_name="s")


def _gather2_body(tA, iA, tB, iB, oA, oB, liA, liB, bA, bB, gsem, wsem):
    wid = lax.axis_index("s") * NC + lax.axis_index("c")
    base = wid * EPW
    pltpu.sync_copy(iA.at[pl.ds(base, EPW)], liA)
    pltpu.sync_copy(iB.at[pl.ds(base, EPW)], liB)

    def body(ci, carry):
        off = ci * GCH
        cpA = pltpu.make_async_copy(tA.at[liA.at[pl.ds(off, GCH)]], bA, gsem)
        cpB = pltpu.make_async_copy(tB.at[liB.at[pl.ds(off, GCH)]], bB, gsem)
        cpA.start()
        cpB.start()
        cpA.wait()
        cpB.wait()
        wA = pltpu.make_async_copy(bA, oA.at[pl.ds(base + off, GCH)], wsem)
        wB = pltpu.make_async_copy(bB, oB.at[pl.ds(base + off, GCH)], wsem)
        wA.start()
        wB.start()
        wA.wait()
        wB.wait()
        return carry

    lax.fori_loop(0, NGC, body, 0)


def _gather2(tableA, idxA, tableB, idxB):
    f = functools.partial(
        pl.kernel,
        out_type=[jax.ShapeDtypeStruct((E, D), F32)] * 2,
        mesh=_sc_mesh,
        scratch_types=[
            pltpu.VMEM((EPW,), jnp.int32),
            pltpu.VMEM((EPW,), jnp.int32),
            pltpu.VMEM((GCH, D), F32),
            pltpu.VMEM((GCH, D), F32),
            pltpu.SemaphoreType.DMA,
            pltpu.SemaphoreType.DMA,
        ],
    )(_gather2_body)
    return f(tableA, idxA, tableB, idxB)


NPAD = 10240
NPT = NPAD // NS


SHALF = 5120
SACC = SHALF + 128
EPT = E // NS
NCT = EPT // GCH
DPT = SHALF // NS


def _scatter_body(arr, dstf, oP, idxb, adjb, bufA, zb, ash, lsem, ssem):
    cid = lax.axis_index("c")
    sid = lax.axis_index("s")
    lo = cid * SHALF
    base = sid * EPT

    def zrow(r, carry):
        for j in range(D // 16):
            zb[r, pl.ds(16 * j, 16)] = jnp.zeros((16,), F32)
        return carry

    lax.fori_loop(0, 128, zrow, 0)

    def zcopy(k, carry):
        @pl.when(lax.rem(k, NS) == sid)
        def _():
            pltpu.sync_copy(zb, ash.at[pl.ds(k * 128, 128)])
        return carry

    lax.fori_loop(0, SACC // 128, zcopy, 0)
    plsc.subcore_barrier()

    def body(c, carry):
        off = base + c * GCH
        pltpu.sync_copy(arr.at[pl.ds(off, GCH)], bufA)
        pltpu.sync_copy(dstf.at[pl.ds(off, GCH)], idxb)
        for v in range(GCH // 16):
            sl = pl.ds(16 * v, 16)
            loc = idxb[sl] - lo
            ok = (loc >= 0) & (loc < SHALF)
            adjb[sl] = jnp.where(ok, loc, SHALF)
        pltpu.sync_copy(bufA, ash.at[adjb], add=True)
        return carry

    lax.fori_loop(0, NCT, body, 0)
    plsc.subcore_barrier()

    pltpu.sync_copy(ash.at[pl.ds(sid * DPT, DPT)],
                    oP.at[cid].at[pl.ds(sid * DPT, DPT)])


def _scatter_accumulate(arr, dstf):
    f = functools.partial(
        pl.kernel,
        out_type=jax.ShapeDtypeStruct((NC, SHALF, D), F32),
        mesh=_sc_mesh,
        scratch_types=[
            pltpu.VMEM((GCH,), jnp.int32),
            pltpu.VMEM((GCH,), jnp.int32),
            pltpu.VMEM((GCH, D), F32),
            pltpu.VMEM((128, D), F32),
            pltpu.VMEM_SHARED((SACC, D), F32),
            pltpu.SemaphoreType.DMA,
            pltpu.SemaphoreType.DMA,
        ],
    )(_scatter_body)
    return f(arr, dstf)


def kernel(x, edge_index, edge_attr, W_l, W_r, W_e, att, bias,
           ln_gamma, ln_beta, mlp_W, mlp_b):
    src = edge_index[0].astype(jnp.int32)
    dst = edge_index[1].astype(jnp.int32)

    att_flat = att.reshape(1, D).astype(F32)
    cc = jnp.arange(D, dtype=jnp.int32)
    hh = jnp.arange(16, dtype=jnp.int32)
    Sh = (cc[:, None] // C == hh[None, :]).astype(F32)
    mask16 = (hh < HEADS).astype(F32).reshape(1, 16)
    R16 = (hh[:, None] == cc[None, :] // C).astype(F32)
    bias_row = bias.reshape(1, D).astype(F32)
    gs = ln_gamma[:D].reshape(1, D).astype(F32)
    gd = ln_gamma[D:].reshape(1, D).astype(F32)
    bs = ln_beta[:D].reshape(1, D).astype(F32)
    bd = ln_beta[D:].reshape(1, D).astype(F32)
    Wt = mlp_W[:D].astype(F32)
    Wb = mlp_W[D:].astype(F32)
    mb = mlp_b.reshape(1, D).astype(F32)

    xl, xr = _project_nodes(x, W_l, W_r)
    gxl, gxr = _gather2(xl, src, xr, dst)
    m, exr = _edge_messages(edge_attr, gxl, gxr, W_e, att_flat, Sh,
                            mask16, R16)
    mP = _scatter_accumulate(m, dst)
    eP = _scatter_accumulate(exr, dst)
    out_pad = _normalize_nodes(mP, eP, bias_row)
    out = out_pad[:N]
    gos, god = _gather2(out_pad, src, out_pad, dst)
    new_edge_attr = _edge_mlp(gos, god, edge_attr, Wt, Wb, gs, gd, bs, bd, mb)
    return out, new_edge_attr


# speedup vs baseline: 23.7221x; 1.2916x over previous
import functools

import jax
import jax.numpy as jnp
from jax import lax
from jax.experimental import pallas as pl
from jax.experimental.pallas import tpu as pltpu
from jax.experimental.pallas import tpu_sc as plsc

F32 = jnp.float32

N = 10000
E = 320000
HEADS = 4
C = 32
D = 128
NEG_SLOPE = 0.2

BN = 1000
BE = 2000


def _k1_body(x_ref, wl_ref, wr_ref, xl_ref, xr_ref):
    xv = x_ref[...]
    xl_ref[...] = jnp.dot(xv, wl_ref[...], preferred_element_type=F32)
    xr_ref[...] = jnp.dot(xv, wr_ref[...], preferred_element_type=F32)


def _project_nodes(x, W_l, W_r):
    return pl.pallas_call(
        _k1_body,
        grid=(N // BN,),
        in_specs=[
            pl.BlockSpec((BN, D), lambda i: (i, 0)),
            pl.BlockSpec((D, D), lambda i: (0, 0)),
            pl.BlockSpec((D, D), lambda i: (0, 0)),
        ],
        out_specs=[pl.BlockSpec((BN, D), lambda i: (i, 0))] * 2,
        out_shape=[jax.ShapeDtypeStruct((N, D), F32)] * 2,
    )(x, W_l, W_r)


def _k3_body(ea_ref, gxl_ref, gxr_ref, we_ref, attf_ref, sh_ref, mask_ref,
             r16_ref, m_ref, exr_ref):
    e = jnp.dot(ea_ref[...], we_ref[...], preferred_element_type=F32)
    s = gxl_ref[...] + gxr_ref[...] + e
    h = jnp.where(s >= 0, s, NEG_SLOPE * s)
    hm = h * attf_ref[...]
    lp = jnp.dot(hm, sh_ref[...], preferred_element_type=F32)
    ex = jnp.exp(lp) * mask_ref[...]
    exr = jnp.dot(ex, r16_ref[...], preferred_element_type=F32)
    exr_ref[...] = exr
    m_ref[...] = exr * gxl_ref[...]


def _edge_messages(edge_attr, gxl, gxr, W_e, att_flat, Sh, mask16, R16):
    return pl.pallas_call(
        _k3_body,
        grid=(E // BE,),
        in_specs=[
            pl.BlockSpec((BE, D), lambda i: (i, 0)),
            pl.BlockSpec((BE, D), lambda i: (i, 0)),
            pl.BlockSpec((BE, D), lambda i: (i, 0)),
            pl.BlockSpec((D, D), lambda i: (0, 0)),
            pl.BlockSpec((1, D), lambda i: (0, 0)),
            pl.BlockSpec((D, 16), lambda i: (0, 0)),
            pl.BlockSpec((1, 16), lambda i: (0, 0)),
            pl.BlockSpec((16, D), lambda i: (0, 0)),
        ],
        out_specs=[
            pl.BlockSpec((BE, D), lambda i: (i, 0)),
            pl.BlockSpec((BE, D), lambda i: (i, 0)),
        ],
        out_shape=[
            jax.ShapeDtypeStruct((E, D), F32),
            jax.ShapeDtypeStruct((E, D), F32),
        ],
    )(edge_attr, gxl, gxr, W_e, att_flat, Sh, mask16, R16)


def _k6_body(mp_ref, ep_ref, bias_ref, out_ref):
    out_ref[...] = mp_ref[0] / (ep_ref[0] + 1e-16) + bias_ref[...]


def _normalize_nodes(mP, eP, bias_row):
    return pl.pallas_call(
        _k6_body,
        grid=(16,),
        in_specs=[
            pl.BlockSpec((1, 640, D), lambda i: (i // 8, i % 8, 0)),
            pl.BlockSpec((1, 640, D), lambda i: (i // 8, i % 8, 0)),
            pl.BlockSpec((1, D), lambda i: (0, 0)),
        ],
        out_specs=pl.BlockSpec((640, D), lambda i: (i, 0)),
        out_shape=jax.ShapeDtypeStruct((10240, D), F32),
    )(mP, eP, bias_row)


def _k8_body(gos_ref, god_ref, ea_ref, wt_ref, wb_ref, gs_ref, gd_ref,
             bs_ref, bd_ref, mb_ref, out_ref):
    gos = gos_ref[...]
    god = god_ref[...]
    mu = (jnp.sum(gos, axis=1, keepdims=True)
          + jnp.sum(god, axis=1, keepdims=True)) / (2 * D)
    cs = gos - mu
    cd = god - mu
    var = (jnp.sum(cs * cs, axis=1, keepdims=True)
           + jnp.sum(cd * cd, axis=1, keepdims=True)) / (2 * D)
    rstd = lax.rsqrt(var + 1e-5)
    ns = cs * rstd * gs_ref[...] + bs_ref[...]
    nd = cd * rstd * gd_ref[...] + bd_ref[...]
    ns = jnp.maximum(ns, 0.0)
    nd = jnp.maximum(nd, 0.0)
    delta = (jnp.dot(ns, wt_ref[...], preferred_element_type=F32)
             + jnp.dot(nd, wb_ref[...], preferred_element_type=F32)
             + mb_ref[...])
    out_ref[...] = ea_ref[...] + delta


def _edge_mlp(gos, god, edge_attr, Wt, Wb, gs, gd, bs, bd, mb):
    return pl.pallas_call(
        _k8_body,
        grid=(E // BE,),
        in_specs=[
            pl.BlockSpec((BE, D), lambda i: (i, 0)),
            pl.BlockSpec((BE, D), lambda i: (i, 0)),
            pl.BlockSpec((BE, D), lambda i: (i, 0)),
            pl.BlockSpec((D, D), lambda i: (0, 0)),
            pl.BlockSpec((D, D), lambda i: (0, 0)),
            pl.BlockSpec((1, D), lambda i: (0, 0)),
            pl.BlockSpec((1, D), lambda i: (0, 0)),
            pl.BlockSpec((1, D), lambda i: (0, 0)),
            pl.BlockSpec((1, D), lambda i: (0, 0)),
            pl.BlockSpec((1, D), lambda i: (0, 0)),
        ],
        out_specs=pl.BlockSpec((BE, D), lambda i: (i, 0)),
        out_shape=jax.ShapeDtypeStruct((E, D), F32),
    )(gos, god, edge_attr, Wt, Wb, gs, gd, bs, bd, mb)


NC = 2
NS = 16
NW = NC * NS
EPW = E // NW
GCH = 80
NGC = EPW // GCH

_sc_mesh = plsc.VectorSubcoreMesh(core_axis_name="c", subcore_axis_name="s")


def _gather2_body(tA, iA, tB, iB, oA, oB, liA, liB,
                  bA0, bA1, bB0, bB1, gsem, wsem):
    wid = lax.axis_index("s") * NC + lax.axis_index("c")
    base = wid * EPW
    pltpu.sync_copy(iA.at[pl.ds(base, EPW)], liA)
    pltpu.sync_copy(iB.at[pl.ds(base, EPW)], liB)
    bufs = ((bA0, bB0), (bA1, bB1))

    def g_start(c, s):
        off = c * GCH
        pltpu.async_copy(tA.at[liA.at[pl.ds(off, GCH)]], bufs[s][0], gsem)
        pltpu.async_copy(tB.at[liB.at[pl.ds(off, GCH)]], bufs[s][1], gsem)

    def g_wait(s):
        pltpu.make_async_copy(tA.at[liA.at[pl.ds(0, GCH)]],
                              bufs[s][0], gsem).wait()
        pltpu.make_async_copy(tB.at[liB.at[pl.ds(0, GCH)]],
                              bufs[s][1], gsem).wait()

    def w_start(c, s):
        off = base + c * GCH
        pltpu.async_copy(bufs[s][0], oA.at[pl.ds(off, GCH)], wsem)
        pltpu.async_copy(bufs[s][1], oB.at[pl.ds(off, GCH)], wsem)

    def w_wait(s):
        pltpu.make_async_copy(bufs[s][0], oA.at[pl.ds(base, GCH)], wsem).wait()
        pltpu.make_async_copy(bufs[s][1], oB.at[pl.ds(base, GCH)], wsem).wait()

    g_start(0, 0)

    def body(p, carry):
        for sub in range(2):
            c = 2 * p + sub
            g_wait(sub)

            @pl.when(c >= 1)
            def _():
                w_wait(1 - sub)

            @pl.when(c + 1 < NGC)
            def _():
                g_start(c + 1, 1 - sub)

            w_start(c, sub)
        return carry

    lax.fori_loop(0, NGC // 2, body, 0)
    g_wait(0)
    w_wait(1)
    w_start(NGC - 1, 0)
    w_wait(0)


def _gather2(tableA, idxA, tableB, idxB):
    f = functools.partial(
        pl.kernel,
        out_type=[jax.ShapeDtypeStruct((E, D), F32)] * 2,
        mesh=_sc_mesh,
        scratch_types=[
            pltpu.VMEM((EPW,), jnp.int32),
            pltpu.VMEM((EPW,), jnp.int32),
            pltpu.VMEM((GCH, D), F32),
            pltpu.VMEM((GCH, D), F32),
            pltpu.VMEM((GCH, D), F32),
            pltpu.VMEM((GCH, D), F32),
            pltpu.SemaphoreType.DMA,
            pltpu.SemaphoreType.DMA,
        ],
    )(_gather2_body)
    return f(tableA, idxA, tableB, idxB)


SHALF = 5120
SACC = SHALF + 128
EPT = E // NS
NCT = EPT // GCH
DPT = SHALF // NS


def _scatter_body(arr, dstf, oP, dall, adj0, adj1, bufA0, bufA1, zb,
                  ash, lsem, ssem):
    cid = lax.axis_index("c")
    sid = lax.axis_index("s")
    lo = cid * SHALF
    base = sid * EPT
    pltpu.sync_copy(dstf.at[pl.ds(base, EPT)], dall)
    bufs = (bufA0, bufA1)
    adjs = (adj0, adj1)

    def zrow(r, carry):
        for j in range(D // 16):
            zb[r, pl.ds(16 * j, 16)] = jnp.zeros((16,), F32)
        return carry

    lax.fori_loop(0, 128, zrow, 0)

    def zcopy(k, carry):
        @pl.when(lax.rem(k, NS) == sid)
        def _():
            pltpu.sync_copy(zb, ash.at[pl.ds(k * 128, 128)])
        return carry

    lax.fori_loop(0, SACC // 128, zcopy, 0)
    plsc.subcore_barrier()

    def l_start(c, s):
        pltpu.async_copy(arr.at[pl.ds(base + c * GCH, GCH)], bufs[s], lsem)

    def l_wait(s):
        pltpu.make_async_copy(arr.at[pl.ds(base, GCH)], bufs[s], lsem).wait()

    def s_start(s):
        pltpu.async_copy(bufs[s], ash.at[adjs[s]], ssem, add=True)

    def s_wait(s):
        pltpu.make_async_copy(bufs[s], ash.at[adjs[s]], ssem).wait()

    l_start(0, 0)

    def body(p, carry):
        for sub in range(2):
            c = 2 * p + sub
            l_wait(sub)
            for v in range(GCH // 16):
                loc = dall[pl.ds(c * GCH + 16 * v, 16)] - lo
                ok = (loc >= 0) & (loc < SHALF)
                adjs[sub][pl.ds(16 * v, 16)] = jnp.where(ok, loc, SHALF)

            @pl.when(c >= 1)
            def _():
                s_wait(1 - sub)

            @pl.when(c + 1 < NCT)
            def _():
                l_start(c + 1, 1 - sub)

            s_start(sub)
        return carry

    lax.fori_loop(0, NCT // 2, body, 0)
    s_wait(1)
    plsc.subcore_barrier()

    pltpu.sync_copy(ash.at[pl.ds(sid * DPT, DPT)],
                    oP.at[cid].at[pl.ds(sid * DPT, DPT)])


def _scatter_accumulate(arr, dstf):
    f = functools.partial(
        pl.kernel,
        out_type=jax.ShapeDtypeStruct((NC, SHALF, D), F32),
        mesh=_sc_mesh,
        scratch_types=[
            pltpu.VMEM((EPT,), jnp.int32),
            pltpu.VMEM((GCH,), jnp.int32),
            pltpu.VMEM((GCH,), jnp.int32),
            pltpu.VMEM((GCH, D), F32),
            pltpu.VMEM((GCH, D), F32),
            pltpu.VMEM((128, D), F32),
            pltpu.VMEM_SHARED((SACC, D), F32),
            pltpu.SemaphoreType.DMA,
            pltpu.SemaphoreType.DMA,
        ],
    )(_scatter_body)
    return f(arr, dstf)


def kernel(x, edge_index, edge_attr, W_l, W_r, W_e, att, bias,
           ln_gamma, ln_beta, mlp_W, mlp_b):
    src = edge_index[0].astype(jnp.int32)
    dst = edge_index[1].astype(jnp.int32)

    att_flat = att.reshape(1, D).astype(F32)
    cc = jnp.arange(D, dtype=jnp.int32)
    hh = jnp.arange(16, dtype=jnp.int32)
    Sh = (cc[:, None] // C == hh[None, :]).astype(F32)
    mask16 = (hh < HEADS).astype(F32).reshape(1, 16)
    R16 = (hh[:, None] == cc[None, :] // C).astype(F32)
    bias_row = bias.reshape(1, D).astype(F32)
    gs = ln_gamma[:D].reshape(1, D).astype(F32)
    gd = ln_gamma[D:].reshape(1, D).astype(F32)
    bs = ln_beta[:D].reshape(1, D).astype(F32)
    bd = ln_beta[D:].reshape(1, D).astype(F32)
    Wt = mlp_W[:D].astype(F32)
    Wb = mlp_W[D:].astype(F32)
    mb = mlp_b.reshape(1, D).astype(F32)

    xl, xr = _project_nodes(x, W_l, W_r)
    gxl, gxr = _gather2(xl, src, xr, dst)
    m, exr = _edge_messages(edge_attr, gxl, gxr, W_e, att_flat, Sh,
                            mask16, R16)
    mP = _scatter_accumulate(m, dst)
    eP = _scatter_accumulate(exr, dst)
    out_pad = _normalize_nodes(mP, eP, bias_row)
    out = out_pad[:N]
    gos, god = _gather2(out_pad, src, out_pad, dst)
    new_edge_attr = _edge_mlp(gos, god, edge_attr, Wt, Wb, gs, gd, bs, bd, mb)
    return out, new_edge_attr


# speedup vs baseline: 25.2644x; 1.0650x over previous
import functools

import jax
import jax.numpy as jnp
from jax import lax
from jax.experimental import pallas as pl
from jax.experimental.pallas import tpu as pltpu
from jax.experimental.pallas import tpu_sc as plsc

F32 = jnp.float32

N = 10000
E = 320000
HEADS = 4
C = 32
D = 128
NEG_SLOPE = 0.2

BN = 2000
BE = 8000


def _k1_body(x_ref, wl_ref, wr_ref, xl_ref, xr_ref):
    xv = x_ref[...]
    xl_ref[...] = jnp.dot(xv, wl_ref[...], preferred_element_type=F32)
    xr_ref[...] = jnp.dot(xv, wr_ref[...], preferred_element_type=F32)


def _project_nodes(x, W_l, W_r):
    return pl.pallas_call(
        _k1_body,
        grid=(N // BN,),
        in_specs=[
            pl.BlockSpec((BN, D), lambda i: (i, 0)),
            pl.BlockSpec((D, D), lambda i: (0, 0)),
            pl.BlockSpec((D, D), lambda i: (0, 0)),
        ],
        out_specs=[pl.BlockSpec((BN, D), lambda i: (i, 0))] * 2,
        out_shape=[jax.ShapeDtypeStruct((N, D), F32)] * 2,
    )(x, W_l, W_r)


def _k3_body(ea_ref, gxl_ref, gxr_ref, we_ref, attf_ref, sh_ref, mask_ref,
             r16_ref, m_ref, exr_ref):
    e = jnp.dot(ea_ref[...], we_ref[...], preferred_element_type=F32)
    s = gxl_ref[...] + gxr_ref[...] + e
    h = jnp.where(s >= 0, s, NEG_SLOPE * s)
    hm = h * attf_ref[...]
    lp = jnp.dot(hm, sh_ref[...], preferred_element_type=F32)
    ex = jnp.exp(lp) * mask_ref[...]
    exr = jnp.dot(ex, r16_ref[...], preferred_element_type=F32)
    exr_ref[...] = exr
    m_ref[...] = exr * gxl_ref[...]


def _edge_messages(edge_attr, gxl, gxr, W_e, att_flat, Sh, mask16, R16):
    return pl.pallas_call(
        _k3_body,
        grid=(E // BE,),
        in_specs=[
            pl.BlockSpec((BE, D), lambda i: (i, 0)),
            pl.BlockSpec((BE, D), lambda i: (i, 0)),
            pl.BlockSpec((BE, D), lambda i: (i, 0)),
            pl.BlockSpec((D, D), lambda i: (0, 0)),
            pl.BlockSpec((1, D), lambda i: (0, 0)),
            pl.BlockSpec((D, 16), lambda i: (0, 0)),
            pl.BlockSpec((1, 16), lambda i: (0, 0)),
            pl.BlockSpec((16, D), lambda i: (0, 0)),
        ],
        out_specs=[
            pl.BlockSpec((BE, D), lambda i: (i, 0)),
            pl.BlockSpec((BE, D), lambda i: (i, 0)),
        ],
        out_shape=[
            jax.ShapeDtypeStruct((E, D), F32),
            jax.ShapeDtypeStruct((E, D), F32),
        ],
    )(edge_attr, gxl, gxr, W_e, att_flat, Sh, mask16, R16)


def _k6_body(mp_ref, ep_ref, bias_ref, out_ref):
    out_ref[...] = mp_ref[0] / (ep_ref[0] + 1e-16) + bias_ref[...]


def _normalize_nodes(mP, eP, bias_row):
    return pl.pallas_call(
        _k6_body,
        grid=(16,),
        in_specs=[
            pl.BlockSpec((1, 640, D), lambda i: (i // 8, i % 8, 0)),
            pl.BlockSpec((1, 640, D), lambda i: (i // 8, i % 8, 0)),
            pl.BlockSpec((1, D), lambda i: (0, 0)),
        ],
        out_specs=pl.BlockSpec((640, D), lambda i: (i, 0)),
        out_shape=jax.ShapeDtypeStruct((10240, D), F32),
    )(mP, eP, bias_row)


def _k8_body(gos_ref, god_ref, ea_ref, wt_ref, wb_ref, gs_ref, gd_ref,
             bs_ref, bd_ref, mb_ref, out_ref):
    gos = gos_ref[...]
    god = god_ref[...]
    mu = (jnp.sum(gos, axis=1, keepdims=True)
          + jnp.sum(god, axis=1, keepdims=True)) / (2 * D)
    cs = gos - mu
    cd = god - mu
    var = (jnp.sum(cs * cs, axis=1, keepdims=True)
           + jnp.sum(cd * cd, axis=1, keepdims=True)) / (2 * D)
    rstd = lax.rsqrt(var + 1e-5)
    ns = cs * rstd * gs_ref[...] + bs_ref[...]
    nd = cd * rstd * gd_ref[...] + bd_ref[...]
    ns = jnp.maximum(ns, 0.0)
    nd = jnp.maximum(nd, 0.0)
    delta = (jnp.dot(ns, wt_ref[...], preferred_element_type=F32)
             + jnp.dot(nd, wb_ref[...], preferred_element_type=F32)
             + mb_ref[...])
    out_ref[...] = ea_ref[...] + delta


def _edge_mlp(gos, god, edge_attr, Wt, Wb, gs, gd, bs, bd, mb):
    return pl.pallas_call(
        _k8_body,
        grid=(E // BE,),
        in_specs=[
            pl.BlockSpec((BE, D), lambda i: (i, 0)),
            pl.BlockSpec((BE, D), lambda i: (i, 0)),
            pl.BlockSpec((BE, D), lambda i: (i, 0)),
            pl.BlockSpec((D, D), lambda i: (0, 0)),
            pl.BlockSpec((D, D), lambda i: (0, 0)),
            pl.BlockSpec((1, D), lambda i: (0, 0)),
            pl.BlockSpec((1, D), lambda i: (0, 0)),
            pl.BlockSpec((1, D), lambda i: (0, 0)),
            pl.BlockSpec((1, D), lambda i: (0, 0)),
            pl.BlockSpec((1, D), lambda i: (0, 0)),
        ],
        out_specs=pl.BlockSpec((BE, D), lambda i: (i, 0)),
        out_shape=jax.ShapeDtypeStruct((E, D), F32),
    )(gos, god, edge_attr, Wt, Wb, gs, gd, bs, bd, mb)


NC = 2
NS = 16
NW = NC * NS
EPW = E // NW
GCH = 80
NGC = EPW // GCH

_sc_mesh = plsc.VectorSubcoreMesh(core_axis_name="c", subcore_axis_name="s")


def _gather2_body(tA, iA, tB, iB, oA, oB, liA, liB,
                  bA0, bA1, bB0, bB1, gsem, wsem):
    wid = lax.axis_index("s") * NC + lax.axis_index("c")
    base = wid * EPW
    pltpu.sync_copy(iA.at[pl.ds(base, EPW)], liA)
    pltpu.sync_copy(iB.at[pl.ds(base, EPW)], liB)
    bufs = ((bA0, bB0), (bA1, bB1))

    def g_start(c, s):
        off = c * GCH
        pltpu.async_copy(tA.at[liA.at[pl.ds(off, GCH)]], bufs[s][0], gsem)
        pltpu.async_copy(tB.at[liB.at[pl.ds(off, GCH)]], bufs[s][1], gsem)

    def g_wait(s):
        pltpu.make_async_copy(tA.at[liA.at[pl.ds(0, GCH)]],
                              bufs[s][0], gsem).wait()
        pltpu.make_async_copy(tB.at[liB.at[pl.ds(0, GCH)]],
                              bufs[s][1], gsem).wait()

    def w_start(c, s):
        off = base + c * GCH
        pltpu.async_copy(bufs[s][0], oA.at[pl.ds(off, GCH)], wsem)
        pltpu.async_copy(bufs[s][1], oB.at[pl.ds(off, GCH)], wsem)

    def w_wait(s):
        pltpu.make_async_copy(bufs[s][0], oA.at[pl.ds(base, GCH)], wsem).wait()
        pltpu.make_async_copy(bufs[s][1], oB.at[pl.ds(base, GCH)], wsem).wait()

    g_start(0, 0)

    def body(p, carry):
        for sub in range(2):
            c = 2 * p + sub
            g_wait(sub)

            @pl.when(c >= 1)
            def _():
                w_wait(1 - sub)

            @pl.when(c + 1 < NGC)
            def _():
                g_start(c + 1, 1 - sub)

            w_start(c, sub)
        return carry

    lax.fori_loop(0, NGC // 2, body, 0)
    g_wait(0)
    w_wait(1)
    w_start(NGC - 1, 0)
    w_wait(0)


def _gather2(tableA, idxA, tableB, idxB):
    f = functools.partial(
        pl.kernel,
        out_type=[jax.ShapeDtypeStruct((E, D), F32)] * 2,
        mesh=_sc_mesh,
        scratch_types=[
            pltpu.VMEM((EPW,), jnp.int32),
            pltpu.VMEM((EPW,), jnp.int32),
            pltpu.VMEM((GCH, D), F32),
            pltpu.VMEM((GCH, D), F32),
            pltpu.VMEM((GCH, D), F32),
            pltpu.VMEM((GCH, D), F32),
            pltpu.SemaphoreType.DMA,
            pltpu.SemaphoreType.DMA,
        ],
    )(_gather2_body)
    return f(tableA, idxA, tableB, idxB)


SHALF = 5120
SACC = SHALF + 128
EPT = E // NS
NCT = EPT // GCH
DPT = SHALF // NS


def _scatter_body(arr, dstf, oP, dall, adj0, adj1, bufA0, bufA1, zb,
                  ash, lsem, ssem):
    cid = lax.axis_index("c")
    sid = lax.axis_index("s")
    lo = cid * SHALF
    base = sid * EPT
    pltpu.sync_copy(dstf.at[pl.ds(base, EPT)], dall)
    bufs = (bufA0, bufA1)
    adjs = (adj0, adj1)

    def zrow(r, carry):
        for j in range(D // 16):
            zb[r, pl.ds(16 * j, 16)] = jnp.zeros((16,), F32)
        return carry

    lax.fori_loop(0, 128, zrow, 0)

    def zcopy(k, carry):
        @pl.when(lax.rem(k, NS) == sid)
        def _():
            pltpu.sync_copy(zb, ash.at[pl.ds(k * 128, 128)])
        return carry

    lax.fori_loop(0, SACC // 128, zcopy, 0)
    plsc.subcore_barrier()

    def l_start(c, s):
        pltpu.async_copy(arr.at[pl.ds(base + c * GCH, GCH)], bufs[s], lsem)

    def l_wait(s):
        pltpu.make_async_copy(arr.at[pl.ds(base, GCH)], bufs[s], lsem).wait()

    def s_start(s):
        pltpu.async_copy(bufs[s], ash.at[adjs[s]], ssem, add=True)

    def s_wait(s):
        pltpu.make_async_copy(bufs[s], ash.at[adjs[s]], ssem).wait()

    l_start(0, 0)

    def body(p, carry):
        for sub in range(2):
            c = 2 * p + sub
            l_wait(sub)
            for v in range(GCH // 16):
                loc = dall[pl.ds(c * GCH + 16 * v, 16)] - lo
                ok = (loc >= 0) & (loc < SHALF)
                adjs[sub][pl.ds(16 * v, 16)] = jnp.where(ok, loc, SHALF)

            @pl.when(c >= 1)
            def _():
                s_wait(1 - sub)

            @pl.when(c + 1 < NCT)
            def _():
                l_start(c + 1, 1 - sub)

            s_start(sub)
        return carry

    lax.fori_loop(0, NCT // 2, body, 0)
    s_wait(1)
    plsc.subcore_barrier()

    pltpu.sync_copy(ash.at[pl.ds(sid * DPT, DPT)],
                    oP.at[cid].at[pl.ds(sid * DPT, DPT)])


def _scatter_accumulate(arr, dstf):
    f = functools.partial(
        pl.kernel,
        out_type=jax.ShapeDtypeStruct((NC, SHALF, D), F32),
        mesh=_sc_mesh,
        scratch_types=[
            pltpu.VMEM((EPT,), jnp.int32),
            pltpu.VMEM((GCH,), jnp.int32),
            pltpu.VMEM((GCH,), jnp.int32),
            pltpu.VMEM((GCH, D), F32),
            pltpu.VMEM((GCH, D), F32),
            pltpu.VMEM((128, D), F32),
            pltpu.VMEM_SHARED((SACC, D), F32),
            pltpu.SemaphoreType.DMA,
            pltpu.SemaphoreType.DMA,
        ],
    )(_scatter_body)
    return f(arr, dstf)


def kernel(x, edge_index, edge_attr, W_l, W_r, W_e, att, bias,
           ln_gamma, ln_beta, mlp_W, mlp_b):
    src = edge_index[0].astype(jnp.int32)
    dst = edge_index[1].astype(jnp.int32)

    att_flat = att.reshape(1, D).astype(F32)
    cc = jnp.arange(D, dtype=jnp.int32)
    hh = jnp.arange(16, dtype=jnp.int32)
    Sh = (cc[:, None] // C == hh[None, :]).astype(F32)
    mask16 = (hh < HEADS).astype(F32).reshape(1, 16)
    R16 = (hh[:, None] == cc[None, :] // C).astype(F32)
    bias_row = bias.reshape(1, D).astype(F32)
    gs = ln_gamma[:D].reshape(1, D).astype(F32)
    gd = ln_gamma[D:].reshape(1, D).astype(F32)
    bs = ln_beta[:D].reshape(1, D).astype(F32)
    bd = ln_beta[D:].reshape(1, D).astype(F32)
    Wt = mlp_W[:D].astype(F32)
    Wb = mlp_W[D:].astype(F32)
    mb = mlp_b.reshape(1, D).astype(F32)

    xl, xr = _project_nodes(x, W_l, W_r)
    gxl, gxr = _gather2(xl, src, xr, dst)
    m, exr = _edge_messages(edge_attr, gxl, gxr, W_e, att_flat, Sh,
                            mask16, R16)
    mP = _scatter_accumulate(m, dst)
    eP = _scatter_accumulate(exr, dst)
    out_pad = _normalize_nodes(mP, eP, bias_row)
    out = out_pad[:N]
    gos, god = _gather2(out_pad, src, out_pad, dst)
    new_edge_attr = _edge_mlp(gos, god, edge_attr, Wt, Wb, gs, gd, bs, bd, mb)
    return out, new_edge_attr


# speedup vs baseline: 27.2725x; 1.0795x over previous
import functools

import jax
import jax.numpy as jnp
from jax import lax
from jax.experimental import pallas as pl
from jax.experimental.pallas import tpu as pltpu
from jax.experimental.pallas import tpu_sc as plsc

F32 = jnp.float32

N = 10000
E = 320000
HEADS = 4
C = 32
D = 128
NEG_SLOPE = 0.2

BN = 2000
BE = 8000


def _k1_body(x_ref, wl_ref, wr_ref, xl_ref, xr_ref):
    xv = x_ref[...]
    xl_ref[...] = jnp.dot(xv, wl_ref[...], preferred_element_type=F32)
    xr_ref[...] = jnp.dot(xv, wr_ref[...], preferred_element_type=F32)


def _project_nodes(x, W_l, W_r):
    return pl.pallas_call(
        _k1_body,
        grid=(N // BN,),
        in_specs=[
            pl.BlockSpec((BN, D), lambda i: (i, 0)),
            pl.BlockSpec((D, D), lambda i: (0, 0)),
            pl.BlockSpec((D, D), lambda i: (0, 0)),
        ],
        out_specs=[pl.BlockSpec((BN, D), lambda i: (i, 0))] * 2,
        out_shape=[jax.ShapeDtypeStruct((N, D), F32)] * 2,
    )(x, W_l, W_r)


def _k3_body(ea_ref, gxl_ref, gxr_ref, we_ref, attf_ref, sh_ref, mask_ref,
             r16_ref, m_ref, exr_ref):
    e = jnp.dot(ea_ref[...], we_ref[...], preferred_element_type=F32)
    s = gxl_ref[...] + gxr_ref[...] + e
    h = jnp.where(s >= 0, s, NEG_SLOPE * s)
    hm = h * attf_ref[...]
    lp = jnp.dot(hm, sh_ref[...], preferred_element_type=F32)
    ex = jnp.exp(lp) * mask_ref[...]
    exr = jnp.dot(ex, r16_ref[...], preferred_element_type=F32)
    exr_ref[...] = exr
    m_ref[...] = exr * gxl_ref[...]


def _edge_messages(edge_attr, gxl, gxr, W_e, att_flat, Sh, mask16, R16):
    return pl.pallas_call(
        _k3_body,
        grid=(E // BE,),
        in_specs=[
            pl.BlockSpec((BE, D), lambda i: (i, 0)),
            pl.BlockSpec((BE, D), lambda i: (i, 0)),
            pl.BlockSpec((BE, D), lambda i: (i, 0)),
            pl.BlockSpec((D, D), lambda i: (0, 0)),
            pl.BlockSpec((1, D), lambda i: (0, 0)),
            pl.BlockSpec((D, 16), lambda i: (0, 0)),
            pl.BlockSpec((1, 16), lambda i: (0, 0)),
            pl.BlockSpec((16, D), lambda i: (0, 0)),
        ],
        out_specs=[
            pl.BlockSpec((BE, D), lambda i: (i, 0)),
            pl.BlockSpec((BE, D), lambda i: (i, 0)),
        ],
        out_shape=[
            jax.ShapeDtypeStruct((E, D), F32),
            jax.ShapeDtypeStruct((E, D), F32),
        ],
    )(edge_attr, gxl, gxr, W_e, att_flat, Sh, mask16, R16)


def _k6_body(mp_ref, ep_ref, bias_ref, out_ref):
    out_ref[...] = mp_ref[0] / (ep_ref[0] + 1e-16) + bias_ref[...]


def _normalize_nodes(mP, eP, bias_row):
    return pl.pallas_call(
        _k6_body,
        grid=(16,),
        in_specs=[
            pl.BlockSpec((1, 640, D), lambda i: (i // 8, i % 8, 0)),
            pl.BlockSpec((1, 640, D), lambda i: (i // 8, i % 8, 0)),
            pl.BlockSpec((1, D), lambda i: (0, 0)),
        ],
        out_specs=pl.BlockSpec((640, D), lambda i: (i, 0)),
        out_shape=jax.ShapeDtypeStruct((10240, D), F32),
    )(mP, eP, bias_row)


def _k8_body(gos_ref, god_ref, ea_ref, wt_ref, wb_ref, gs_ref, gd_ref,
             bs_ref, bd_ref, mb_ref, out_ref):
    gos = gos_ref[...]
    god = god_ref[...]
    mu = (jnp.sum(gos, axis=1, keepdims=True)
          + jnp.sum(god, axis=1, keepdims=True)) / (2 * D)
    cs = gos - mu
    cd = god - mu
    var = (jnp.sum(cs * cs, axis=1, keepdims=True)
           + jnp.sum(cd * cd, axis=1, keepdims=True)) / (2 * D)
    rstd = lax.rsqrt(var + 1e-5)
    ns = cs * rstd * gs_ref[...] + bs_ref[...]
    nd = cd * rstd * gd_ref[...] + bd_ref[...]
    ns = jnp.maximum(ns, 0.0)
    nd = jnp.maximum(nd, 0.0)
    delta = (jnp.dot(ns, wt_ref[...], preferred_element_type=F32)
             + jnp.dot(nd, wb_ref[...], preferred_element_type=F32)
             + mb_ref[...])
    out_ref[...] = ea_ref[...] + delta


def _edge_mlp(gos, god, edge_attr, Wt, Wb, gs, gd, bs, bd, mb):
    return pl.pallas_call(
        _k8_body,
        grid=(E // BE,),
        in_specs=[
            pl.BlockSpec((BE, D), lambda i: (i, 0)),
            pl.BlockSpec((BE, D), lambda i: (i, 0)),
            pl.BlockSpec((BE, D), lambda i: (i, 0)),
            pl.BlockSpec((D, D), lambda i: (0, 0)),
            pl.BlockSpec((D, D), lambda i: (0, 0)),
            pl.BlockSpec((1, D), lambda i: (0, 0)),
            pl.BlockSpec((1, D), lambda i: (0, 0)),
            pl.BlockSpec((1, D), lambda i: (0, 0)),
            pl.BlockSpec((1, D), lambda i: (0, 0)),
            pl.BlockSpec((1, D), lambda i: (0, 0)),
        ],
        out_specs=pl.BlockSpec((BE, D), lambda i: (i, 0)),
        out_shape=jax.ShapeDtypeStruct((E, D), F32),
    )(gos, god, edge_attr, Wt, Wb, gs, gd, bs, bd, mb)


NC = 2
NS = 16
NW = NC * NS
EPW = E // NW
GCH = 128
NGF = EPW // GCH
GREM = EPW - NGF * GCH
NCF = 20000 // GCH
SREM = 20000 - NCF * GCH

_sc_mesh = plsc.VectorSubcoreMesh(core_axis_name="c", subcore_axis_name="s")


def _gather2_body(tA, iA, tB, iB, oA, oB, liA, liB,
                  bA0, bA1, bB0, bB1, rA, rB, gsem, wsem):
    wid = lax.axis_index("s") * NC + lax.axis_index("c")
    base = wid * EPW
    pltpu.sync_copy(iA.at[pl.ds(base, EPW)], liA)
    pltpu.sync_copy(iB.at[pl.ds(base, EPW)], liB)
    bufs = ((bA0, bB0), (bA1, bB1))

    def g_start(c, s):
        off = c * GCH
        pltpu.async_copy(tA.at[liA.at[pl.ds(off, GCH)]], bufs[s][0], gsem)
        pltpu.async_copy(tB.at[liB.at[pl.ds(off, GCH)]], bufs[s][1], gsem)

    def g_wait(s):
        pltpu.make_async_copy(tA.at[liA.at[pl.ds(0, GCH)]],
                              bufs[s][0], gsem).wait()
        pltpu.make_async_copy(tB.at[liB.at[pl.ds(0, GCH)]],
                              bufs[s][1], gsem).wait()

    def w_start(c, s):
        off = base + c * GCH
        pltpu.async_copy(bufs[s][0], oA.at[pl.ds(off, GCH)], wsem)
        pltpu.async_copy(bufs[s][1], oB.at[pl.ds(off, GCH)], wsem)

    def w_wait(s):
        pltpu.make_async_copy(bufs[s][0], oA.at[pl.ds(base, GCH)], wsem).wait()
        pltpu.make_async_copy(bufs[s][1], oB.at[pl.ds(base, GCH)], wsem).wait()

    g_start(0, 0)

    def body(p, carry):
        for sub in range(2):
            c = 2 * p + sub
            g_wait(sub)

            @pl.when(c >= 1)
            def _():
                w_wait(1 - sub)

            @pl.when(c + 1 < NGF)
            def _():
                g_start(c + 1, 1 - sub)

            w_start(c, sub)
        return carry

    lax.fori_loop(0, NGF // 2, body, 0)
    w_wait(1)
    roff = NGF * GCH
    pltpu.sync_copy(tA.at[liA.at[pl.ds(roff, GREM)]], rA)
    pltpu.sync_copy(tB.at[liB.at[pl.ds(roff, GREM)]], rB)
    pltpu.sync_copy(rA, oA.at[pl.ds(base + roff, GREM)])
    pltpu.sync_copy(rB, oB.at[pl.ds(base + roff, GREM)])


def _gather2(tableA, idxA, tableB, idxB):
    f = functools.partial(
        pl.kernel,
        out_type=[jax.ShapeDtypeStruct((E, D), F32)] * 2,
        mesh=_sc_mesh,
        scratch_types=[
            pltpu.VMEM((EPW,), jnp.int32),
            pltpu.VMEM((EPW,), jnp.int32),
            pltpu.VMEM((GCH, D), F32),
            pltpu.VMEM((GCH, D), F32),
            pltpu.VMEM((GCH, D), F32),
            pltpu.VMEM((GCH, D), F32),
            pltpu.VMEM((GREM, D), F32),
            pltpu.VMEM((GREM, D), F32),
            pltpu.SemaphoreType.DMA,
            pltpu.SemaphoreType.DMA,
        ],
    )(_gather2_body)
    return f(tableA, idxA, tableB, idxB)


SHALF = 5120
SACC = SHALF + 128
EPT = E // NS
DPT = SHALF // NS


def _scatter_body(arr, dstf, oP, dall, adj0, adj1, radj, bufA0, bufA1,
                  rbuf, zb, ash, lsem, ssem):
    cid = lax.axis_index("c")
    sid = lax.axis_index("s")
    lo = cid * SHALF
    base = sid * EPT
    pltpu.sync_copy(dstf.at[pl.ds(base, EPT)], dall)
    bufs = (bufA0, bufA1)
    adjs = (adj0, adj1)

    def zrow(r, carry):
        for j in range(D // 16):
            zb[r, pl.ds(16 * j, 16)] = jnp.zeros((16,), F32)
        return carry

    lax.fori_loop(0, 128, zrow, 0)

    def zcopy(k, carry):
        @pl.when(lax.rem(k, NS) == sid)
        def _():
            pltpu.sync_copy(zb, ash.at[pl.ds(k * 128, 128)])
        return carry

    lax.fori_loop(0, SACC // 128, zcopy, 0)
    plsc.subcore_barrier()

    def l_start(c, s):
        pltpu.async_copy(arr.at[pl.ds(base + c * GCH, GCH)], bufs[s], lsem)

    def l_wait(s):
        pltpu.make_async_copy(arr.at[pl.ds(base, GCH)], bufs[s], lsem).wait()

    def s_start(s):
        pltpu.async_copy(bufs[s], ash.at[adjs[s]], ssem, add=True)

    def s_wait(s):
        pltpu.make_async_copy(bufs[s], ash.at[adjs[s]], ssem).wait()

    l_start(0, 0)

    def body(p, carry):
        for sub in range(2):
            c = 2 * p + sub
            l_wait(sub)
            for v in range(GCH // 16):
                loc = dall[pl.ds(c * GCH + 16 * v, 16)] - lo
                ok = (loc >= 0) & (loc < SHALF)
                adjs[sub][pl.ds(16 * v, 16)] = jnp.where(ok, loc, SHALF)

            @pl.when(c >= 1)
            def _():
                s_wait(1 - sub)

            @pl.when(c + 1 < NCF)
            def _():
                l_start(c + 1, 1 - sub)

            s_start(sub)
        return carry

    lax.fori_loop(0, NCF // 2, body, 0)
    s_wait(1)
    roff = NCF * GCH
    pltpu.sync_copy(arr.at[pl.ds(base + roff, SREM)], rbuf)
    for v in range(SREM // 16):
        loc = dall[pl.ds(roff + 16 * v, 16)] - lo
        ok = (loc >= 0) & (loc < SHALF)
        radj[pl.ds(16 * v, 16)] = jnp.where(ok, loc, SHALF)
    pltpu.sync_copy(rbuf, ash.at[radj], add=True)
    plsc.subcore_barrier()

    pltpu.sync_copy(ash.at[pl.ds(sid * DPT, DPT)],
                    oP.at[cid].at[pl.ds(sid * DPT, DPT)])


def _scatter_accumulate(arr, dstf):
    f = functools.partial(
        pl.kernel,
        out_type=jax.ShapeDtypeStruct((NC, SHALF, D), F32),
        mesh=_sc_mesh,
        scratch_types=[
            pltpu.VMEM((EPT,), jnp.int32),
            pltpu.VMEM((GCH,), jnp.int32),
            pltpu.VMEM((GCH,), jnp.int32),
            pltpu.VMEM((SREM,), jnp.int32),
            pltpu.VMEM((GCH, D), F32),
            pltpu.VMEM((GCH, D), F32),
            pltpu.VMEM((SREM, D), F32),
            pltpu.VMEM((128, D), F32),
            pltpu.VMEM_SHARED((SACC, D), F32),
            pltpu.SemaphoreType.DMA,
            pltpu.SemaphoreType.DMA,
        ],
    )(_scatter_body)
    return f(arr, dstf)


def kernel(x, edge_index, edge_attr, W_l, W_r, W_e, att, bias,
           ln_gamma, ln_beta, mlp_W, mlp_b):
    src = edge_index[0].astype(jnp.int32)
    dst = edge_index[1].astype(jnp.int32)

    att_flat = att.reshape(1, D).astype(F32)
    cc = jnp.arange(D, dtype=jnp.int32)
    hh = jnp.arange(16, dtype=jnp.int32)
    Sh = (cc[:, None] // C == hh[None, :]).astype(F32)
    mask16 = (hh < HEADS).astype(F32).reshape(1, 16)
    R16 = (hh[:, None] == cc[None, :] // C).astype(F32)
    bias_row = bias.reshape(1, D).astype(F32)
    gs = ln_gamma[:D].reshape(1, D).astype(F32)
    gd = ln_gamma[D:].reshape(1, D).astype(F32)
    bs = ln_beta[:D].reshape(1, D).astype(F32)
    bd = ln_beta[D:].reshape(1, D).astype(F32)
    Wt = mlp_W[:D].astype(F32)
    Wb = mlp_W[D:].astype(F32)
    mb = mlp_b.reshape(1, D).astype(F32)

    xl, xr = _project_nodes(x, W_l, W_r)
    gxl, gxr = _gather2(xl, src, xr, dst)
    m, exr = _edge_messages(edge_attr, gxl, gxr, W_e, att_flat, Sh,
                            mask16, R16)
    mP = _scatter_accumulate(m, dst)
    eP = _scatter_accumulate(exr, dst)
    out_pad = _normalize_nodes(mP, eP, bias_row)
    out = out_pad[:N]
    gos, god = _gather2(out_pad, src, out_pad, dst)
    new_edge_attr = _edge_mlp(gos, god, edge_attr, Wt, Wb, gs, gd, bs, bd, mb)
    return out, new_edge_attr
